# baseline (speedup 1.0000x reference)
import functools

import jax
import jax.numpy as jnp
from jax import lax
from jax.experimental import pallas as pl
from jax.experimental.pallas import tpu as pltpu
from jax.experimental.pallas import tpu_sc as plsc

B, N, K, L = 8, 512, 2048, 128
NC, NS = 2, 16
BPC = B // NC
TPB = NS // BPC
RPT = K // TPB
ACC_ROWS = 2 * BPC * N
STRIPE = ACC_ROWS // NS
SUB = 128
NSUB = RPT // SUB

_mesh = plsc.VectorSubcoreMesh(core_axis_name="c", subcore_axis_name="s")
_f32 = jnp.float32


def _zero_buf(buf):
    zeros = jnp.zeros((16,), jnp.float32)

    def body(r, _):
        for col in range(L // 16):
            buf[r, pl.ds(col * 16, 16)] = zeros
        return 0

    lax.fori_loop(0, SUB, body, 0)


def _load_idx(inds_hbm, idx_v, b, k0, offs):
    for m in range(2):
        for j in range(NSUB):
            pltpu.sync_copy(inds_hbm.at[m, b, pl.ds(k0 + j * SUB, SUB)],
                            idx_v.at[m, j])
    for m in range(2):
        for j in range(NSUB):
            for col in range(SUB // 16):
                sl = pl.ds(col * 16, 16)
                idx_v[m, j, sl] = idx_v[m, j, sl] + offs[m]


def _sc_scatter_body(pred_hbm, inds_hbm, out_hbm,
                     acc_sh, idx_v, rows0, rows1, semZ, semL, semF):
    c = lax.axis_index("c")
    s = lax.axis_index("s")
    bl = s // TPB
    b = c * BPC + bl
    k0 = (s % TPB) * RPT
    q = s * STRIPE
    m = q // (BPC * N)
    g = m * (B * N) + c * (BPC * N) + q % (BPC * N)

    _zero_buf(rows0)
    zs = [pltpu.async_copy(rows0, acc_sh.at[pl.ds(q + t * SUB, SUB)], semZ)
          for t in range(STRIPE // SUB)]
    _load_idx(inds_hbm, idx_v, b, k0, (bl * N, BPC * N + bl * N))
    for h in zs:
        h.wait()
    plsc.subcore_barrier()

    bufs = (rows0, rows1)
    loads = [pltpu.async_copy(pred_hbm.at[b, pl.ds(k0, SUB)], bufs[0], semL)]
    for j in range(NSUB):
        buf = bufs[j % 2]
        loads[j].wait()
        if j + 1 < NSUB:
            loads.append(pltpu.async_copy(
                pred_hbm.at[b, pl.ds(k0 + (j + 1) * SUB, SUB)],
                bufs[(j + 1) % 2], semL))
        pltpu.sync_copy(buf, acc_sh.at[idx_v.at[0, j]], add=True)
        pltpu.sync_copy(buf, acc_sh.at[idx_v.at[1, j]], add=True)

    plsc.subcore_barrier()
    nt = STRIPE // SUB
    hs = [None] * nt
    for t in range(nt):
        if t >= 2:
            hs[t - 2].wait()
        pltpu.sync_copy(acc_sh.at[pl.ds(q + t * SUB, SUB)], bufs[t % 2])
        hs[t] = pltpu.async_copy(bufs[t % 2],
                                 out_hbm.at[pl.ds(g + t * SUB, SUB)], semF)
    for t in range(max(0, nt - 2), nt):
        hs[t].wait()


_DMA = pltpu.SemaphoreType.DMA

_sc_scatter = pl.kernel(
    _sc_scatter_body, mesh=_mesh,
    out_type=[jax.ShapeDtypeStruct((2 * B * N, L), _f32)],
    scratch_types=[
        pltpu.VMEM_SHARED((ACC_ROWS, L), _f32),
        pltpu.VMEM((2, NSUB, SUB), jnp.int32),
        pltpu.VMEM((SUB, L), _f32),
        pltpu.VMEM((SUB, L), _f32),
        _DMA, _DMA, _DMA,
    ],
)


def _onehots(ind_ref):
    ind_s = ind_ref[0, :, 0]
    ind_o = ind_ref[0, :, 1]
    iota_n = jax.lax.broadcasted_iota(jnp.int32, (K, N), 1)
    m_sT = (ind_s[:, None] == iota_n).astype(jnp.bfloat16)
    m_oT = (ind_o[:, None] == iota_n).astype(jnp.bfloat16)
    return m_sT, m_oT


_mm = functools.partial(jnp.dot, preferred_element_type=_f32)


def _tca_body(att_ref, pred_ref, ind_ref, w_pred_ref, w_sp_ref, w_op_ref,
              npred1_ref):
    x = att_ref[0]
    m_sT, m_oT = _onehots(ind_ref)
    a_s = _mm(x, w_sp_ref[0]).astype(jnp.bfloat16)
    a_o = _mm(x, w_op_ref[0]).astype(jnp.bfloat16)
    npred1_ref[0] = jax.nn.relu(_mm(pred_ref[0], w_pred_ref[0])
                                + _mm(m_sT, a_s) + _mm(m_oT, a_o))


def _tcb_body(att_ref, s1_ref, npred1_ref, pred_ref, ind_ref,
              w_obj_ref, w_ps_ref, w_po_ref, w_pred_ref, w_sp_ref, w_op_ref,
              opred_ref, x1_ref):
    att = att_ref[0]
    x1 = jax.nn.relu(_mm(att, w_obj_ref[0]) + _mm(s1_ref[0, 0], w_ps_ref[0])
                     + _mm(s1_ref[1, 0], w_po_ref[0]))
    m_sT, m_oT = _onehots(ind_ref)
    a2s = _mm(x1, w_sp_ref[1]).astype(jnp.bfloat16)
    a2o = _mm(x1, w_op_ref[1]).astype(jnp.bfloat16)
    npred2 = jax.nn.relu(_mm(npred1_ref[0], w_pred_ref[1])
                         + _mm(m_sT, a2s) + _mm(m_oT, a2o)) + pred_ref[0]
    for cc in range(5):
        opred_ref[0, cc] = npred2
    x1_ref[0] = x1


def _tcc_body(att_ref, x1_ref, s2_ref, w_obj_ref, w_ps_ref, w_po_ref,
              oobj_ref):
    obj2 = jax.nn.relu(_mm(x1_ref[0], w_obj_ref[1])
                       + _mm(s2_ref[0, 0], w_ps_ref[1])
                       + _mm(s2_ref[1, 0], w_po_ref[1])) + att_ref[0]
    for cc in range(5):
        oobj_ref[0, cc] = obj2


_w_spec = pl.BlockSpec((2, L, L), lambda i: (0, 0, 0))
_att_spec = pl.BlockSpec((1, N, L), lambda i: (i, 0, 0))
_pred_spec = pl.BlockSpec((1, K, L), lambda i: (i, 0, 0))
_ind_spec = pl.BlockSpec((1, K, 2), lambda i: (i, 0, 0))
_s_spec = pl.BlockSpec((2, 1, N, L), lambda i: (0, i, 0, 0))


def _tc_a(att, pred, rel_ind, w_pred, w_sp, w_op):
    return pl.pallas_call(
        _tca_body,
        grid=(B,),
        in_specs=[_att_spec, _pred_spec, _ind_spec] + [_w_spec] * 3,
        out_specs=[_pred_spec],
        out_shape=[jax.ShapeDtypeStruct((B, K, L), _f32)],
    )(att, pred, rel_ind, w_pred, w_sp, w_op)


def _tc_b(att, s1, npred1, pred, rel_ind, *ws):
    return pl.pallas_call(
        _tcb_body,
        grid=(B,),
        in_specs=[_att_spec, _s_spec, _pred_spec, _pred_spec, _ind_spec]
        + [_w_spec] * 6,
        out_specs=[
            pl.BlockSpec((1, 5, K, L), lambda i: (i, 0, 0, 0)),
            _att_spec,
        ],
        out_shape=[
            jax.ShapeDtypeStruct((B, 5, K, L), _f32),
            jax.ShapeDtypeStruct((B, N, L), _f32),
        ],
    )(att, s1, npred1, pred, rel_ind, *ws)


def _tc_c(att, x1, s2, w_obj, w_ps, w_po):
    return pl.pallas_call(
        _tcc_body,
        grid=(B,),
        in_specs=[_att_spec, _att_spec, _s_spec] + [_w_spec] * 3,
        out_specs=[pl.BlockSpec((1, 5, N, L), lambda i: (i, 0, 0, 0))],
        out_shape=[jax.ShapeDtypeStruct((B, 5, N, L), _f32)],
    )(att, x1, s2, w_obj, w_ps, w_po)


def kernel(b, N_, K_, L_, att_feats, obj_dist, pred_fmap, rel_ind,
           W_obj, W_ps, W_po, W_pred, W_sp, W_op):
    del b, N_, K_, L_, obj_dist
    ws = (W_obj, W_ps, W_po, W_pred, W_sp, W_op)
    inds_t = jnp.transpose(rel_ind, (2, 0, 1))

    (s1,) = _sc_scatter(pred_fmap, inds_t)
    (npred1,) = _tc_a(att_feats, pred_fmap, rel_ind, W_pred, W_sp, W_op)
    (s2,) = _sc_scatter(npred1, inds_t)
    opred, x1 = _tc_b(att_feats, s1.reshape(2, B, N, L), npred1,
                      pred_fmap, rel_ind, *ws)
    (oobj,) = _tc_c(att_feats, x1, s2.reshape(2, B, N, L), W_obj, W_ps, W_po)
    return (oobj.reshape(B * 5, N, L), opred.reshape(B * 5, K, L))


# speedup vs baseline: 1.1004x; 1.1004x over previous
import functools

import jax
import jax.numpy as jnp
from jax import lax
from jax.experimental import pallas as pl
from jax.experimental.pallas import tpu as pltpu
from jax.experimental.pallas import tpu_sc as plsc

B, N, K, L = 8, 512, 2048, 128
NC, NS = 2, 16
BPC = B // NC
TPB = NS // BPC
RPT = K // TPB
ACC_ROWS = 2 * BPC * N
STRIPE = ACC_ROWS // NS
SUB = 128
NSUB = RPT // SUB

_mesh = plsc.VectorSubcoreMesh(core_axis_name="c", subcore_axis_name="s")
_f32 = jnp.float32


def _zero_buf(buf):
    zeros = jnp.zeros((16,), jnp.float32)

    def body(r, _):
        for col in range(L // 16):
            buf[r, pl.ds(col * 16, 16)] = zeros
        return 0

    lax.fori_loop(0, SUB, body, 0)


def _load_idx(inds_hbm, idx_v, b, k0, offs):
    for m in range(2):
        for j in range(NSUB):
            pltpu.sync_copy(inds_hbm.at[m, b, pl.ds(k0 + j * SUB, SUB)],
                            idx_v.at[m, j])
    for m in range(2):
        for j in range(NSUB):
            for col in range(SUB // 16):
                sl = pl.ds(col * 16, 16)
                idx_v[m, j, sl] = idx_v[m, j, sl] + offs[m]


def _sc_scatter_body(pred_hbm, inds_hbm, out_hbm,
                     acc_sh, idx_v, rows0, rows1, semZ, semL, semF):
    c = lax.axis_index("c")
    s = lax.axis_index("s")
    bl = s // TPB
    b = c * BPC + bl
    k0 = (s % TPB) * RPT
    q = s * STRIPE
    m = q // (BPC * N)
    g = m * (B * N) + c * (BPC * N) + q % (BPC * N)

    _zero_buf(rows0)
    zs = [pltpu.async_copy(rows0, acc_sh.at[pl.ds(q + t * SUB, SUB)], semZ)
          for t in range(STRIPE // SUB)]
    _load_idx(inds_hbm, idx_v, b, k0, (bl * N, BPC * N + bl * N))
    for h in zs:
        h.wait()
    plsc.subcore_barrier()

    bufs = (rows0, rows1)
    loads = [pltpu.async_copy(pred_hbm.at[b, pl.ds(k0, SUB)], bufs[0], semL)]
    for j in range(NSUB):
        buf = bufs[j % 2]
        loads[j].wait()
        if j + 1 < NSUB:
            loads.append(pltpu.async_copy(
                pred_hbm.at[b, pl.ds(k0 + (j + 1) * SUB, SUB)],
                bufs[(j + 1) % 2], semL))
        pltpu.sync_copy(buf, acc_sh.at[idx_v.at[0, j]], add=True)
        pltpu.sync_copy(buf, acc_sh.at[idx_v.at[1, j]], add=True)

    plsc.subcore_barrier()
    nt = STRIPE // SUB
    hs = [None] * nt
    for t in range(nt):
        if t >= 2:
            hs[t - 2].wait()
        pltpu.sync_copy(acc_sh.at[pl.ds(q + t * SUB, SUB)], bufs[t % 2])
        hs[t] = pltpu.async_copy(bufs[t % 2],
                                 out_hbm.at[pl.ds(g + t * SUB, SUB)], semF)
    for t in range(max(0, nt - 2), nt):
        hs[t].wait()


_DMA = pltpu.SemaphoreType.DMA

_sc_scatter = pl.kernel(
    _sc_scatter_body, mesh=_mesh,
    out_type=[jax.ShapeDtypeStruct((2 * B * N, L), _f32)],
    scratch_types=[
        pltpu.VMEM_SHARED((ACC_ROWS, L), _f32),
        pltpu.VMEM((2, NSUB, SUB), jnp.int32),
        pltpu.VMEM((SUB, L), _f32),
        pltpu.VMEM((SUB, L), _f32),
        _DMA, _DMA, _DMA,
    ],
)


def _onehots(ind_ref):
    ind_s = ind_ref[0, 0, 0]
    ind_o = ind_ref[1, 0, 0]
    iota_n = jax.lax.broadcasted_iota(jnp.int32, (K, N), 1)
    m_sT = (ind_s[:, None] == iota_n).astype(_f32)
    m_oT = (ind_o[:, None] == iota_n).astype(_f32)
    return m_sT, m_oT


_mm = functools.partial(jnp.dot, preferred_element_type=_f32)


def _tca_body(att_ref, pred_ref, ind_ref, w_pred_ref, w_sp_ref, w_op_ref,
              npred1_ref):
    x = att_ref[0]
    m_sT, m_oT = _onehots(ind_ref)
    a_s = _mm(x, w_sp_ref[0])
    a_o = _mm(x, w_op_ref[0])
    npred1_ref[0] = jax.nn.relu(_mm(pred_ref[0], w_pred_ref[0])
                                + _mm(m_sT, a_s) + _mm(m_oT, a_o))


def _tcb_body(att_ref, s1_ref, npred1_ref, pred_ref, ind_ref,
              w_obj_ref, w_ps_ref, w_po_ref, w_pred_ref, w_sp_ref, w_op_ref,
              opred_ref, x1_ref):
    att = att_ref[0]
    x1 = jax.nn.relu(_mm(att, w_obj_ref[0]) + _mm(s1_ref[0, 0], w_ps_ref[0])
                     + _mm(s1_ref[1, 0], w_po_ref[0]))
    m_sT, m_oT = _onehots(ind_ref)
    a2s = _mm(x1, w_sp_ref[1])
    a2o = _mm(x1, w_op_ref[1])
    npred2 = jax.nn.relu(_mm(npred1_ref[0], w_pred_ref[1])
                         + _mm(m_sT, a2s) + _mm(m_oT, a2o)) + pred_ref[0]
    for cc in range(5):
        opred_ref[0, cc] = npred2
    x1_ref[0] = x1


def _tcc_body(att_ref, x1_ref, s2_ref, w_obj_ref, w_ps_ref, w_po_ref,
              oobj_ref):
    obj2 = jax.nn.relu(_mm(x1_ref[0], w_obj_ref[1])
                       + _mm(s2_ref[0, 0], w_ps_ref[1])
                       + _mm(s2_ref[1, 0], w_po_ref[1])) + att_ref[0]
    for cc in range(5):
        oobj_ref[0, cc] = obj2


_w_spec = pl.BlockSpec((2, L, L), lambda i: (0, 0, 0))
_att_spec = pl.BlockSpec((1, N, L), lambda i: (i, 0, 0))
_pred_spec = pl.BlockSpec((1, K, L), lambda i: (i, 0, 0))
_ind_spec = pl.BlockSpec((2, 1, 1, K), lambda i: (0, i, 0, 0))
_s_spec = pl.BlockSpec((2, 1, N, L), lambda i: (0, i, 0, 0))


def _tc_a(att, pred, rel_ind, w_pred, w_sp, w_op):
    return pl.pallas_call(
        _tca_body,
        grid=(B,),
        in_specs=[_att_spec, _pred_spec, _ind_spec] + [_w_spec] * 3,
        out_specs=[_pred_spec],
        out_shape=[jax.ShapeDtypeStruct((B, K, L), _f32)],
    )(att, pred, rel_ind, w_pred, w_sp, w_op)


def _tc_b(att, s1, npred1, pred, rel_ind, *ws):
    return pl.pallas_call(
        _tcb_body,
        grid=(B,),
        in_specs=[_att_spec, _s_spec, _pred_spec, _pred_spec, _ind_spec]
        + [_w_spec] * 6,
        out_specs=[
            pl.BlockSpec((1, 5, K, L), lambda i: (i, 0, 0, 0)),
            _att_spec,
        ],
        out_shape=[
            jax.ShapeDtypeStruct((B, 5, K, L), _f32),
            jax.ShapeDtypeStruct((B, N, L), _f32),
        ],
    )(att, s1, npred1, pred, rel_ind, *ws)


def _tc_c(att, x1, s2, w_obj, w_ps, w_po):
    return pl.pallas_call(
        _tcc_body,
        grid=(B,),
        in_specs=[_att_spec, _att_spec, _s_spec] + [_w_spec] * 3,
        out_specs=[pl.BlockSpec((1, 5, N, L), lambda i: (i, 0, 0, 0))],
        out_shape=[jax.ShapeDtypeStruct((B, 5, N, L), _f32)],
    )(att, x1, s2, w_obj, w_ps, w_po)


def kernel(b, N_, K_, L_, att_feats, obj_dist, pred_fmap, rel_ind,
           W_obj, W_ps, W_po, W_pred, W_sp, W_op):
    del b, N_, K_, L_, obj_dist
    ws = (W_obj, W_ps, W_po, W_pred, W_sp, W_op)
    inds_t = jnp.transpose(rel_ind, (2, 0, 1))

    (s1,) = _sc_scatter(pred_fmap, inds_t)
    inds4 = inds_t.reshape(2, B, 1, K)
    (npred1,) = _tc_a(att_feats, pred_fmap, inds4, W_pred, W_sp, W_op)
    (s2,) = _sc_scatter(npred1, inds_t)
    opred, x1 = _tc_b(att_feats, s1.reshape(2, B, N, L), npred1,
                      pred_fmap, inds4, *ws)
    (oobj,) = _tc_c(att_feats, x1, s2.reshape(2, B, N, L), W_obj, W_ps, W_po)
    return (oobj.reshape(B * 5, N, L), opred.reshape(B * 5, K, L))
